# baseline (speedup 1.0000x reference)
import jax
import jax.numpy as jnp
from jax import lax
from jax.experimental import pallas as pl
from jax.experimental.pallas import tpu as pltpu
from jax.experimental.pallas import tpu_sc as plsc

_VOCAB = 1000000
_D = 32
_BATCH = 16384
_FIELDS = 26
_N = _BATCH * _FIELDS
_NC = 2
_NS = 16
_NW = _NC * _NS
_BLK = 512
_NBLK = _BATCH // _BLK
_UNITS = _FIELDS * _NBLK
_PER_W = _UNITS // _NW

_TW = 512
_TCOLS = _VOCAB // _TW
_TPW = _TCOLS // _NW
_TEXTRA = _TCOLS - _TPW * _NW
_TAIL = _VOCAB - _TCOLS * _TW


def _transp_body(tabt_hbm, out_hbm, tb0, tb1, ob0, ob1, tbt, obt, i_sem,
                 o_sem):
    wid = lax.axis_index("s") * _NC + lax.axis_index("c")
    base = wid * _TPW
    lanes = lax.broadcasted_iota(jnp.int32, (16,), 0)
    lanes32 = lanes * 32
    rot = [(lanes + k) & 15 for k in range(16)]
    tb = (tb0, tb1)
    ob = (ob0, ob1)

    def transpose(src, dst, ncols):
        def tile_step(t, _):
            c0 = (t % (ncols // 16)) * 16
            d0 = (t // (ncols // 16)) * 16
            colv = lanes + c0
            dv0 = lanes32 + (c0 * 32 + d0)
            for k in range(16):
                rv = rot[k] + d0
                vals = plsc.load_gather(src, [rv, colv])
                plsc.store_scatter(dst, [dv0 + rot[k]], vals)
            return 0

        lax.fori_loop(0, (ncols // 16) * (_D // 16), tile_step, 0)

    def fire_in(c, b):
        pltpu.make_async_copy(
            tabt_hbm.at[:, pl.ds(c * _TW, _TW)], tb[b], i_sem.at[b]).start()

    def wait_in(c, b):
        pltpu.make_async_copy(
            tabt_hbm.at[:, pl.ds(c * _TW, _TW)], tb[b], i_sem.at[b]).wait()

    def fire_out(c, b):
        pltpu.make_async_copy(
            ob[b], out_hbm.at[pl.ds(c * _TW * _D, _TW * _D)], o_sem.at[b]).start()

    def wait_out(c, b):
        pltpu.make_async_copy(
            ob[b], out_hbm.at[pl.ds(c * _TW * _D, _TW * _D)], o_sem.at[b]).wait()

    fire_in(base, 0)

    def outer(i2, _):
        for b in range(2):
            j = i2 * 2 + b
            c = base + j
            wait_in(c, b)

            @pl.when(j + 1 < _TPW)
            def _():
                fire_in(c + 1, 1 - b)

            @pl.when(j >= 2)
            def _():
                wait_out(c - 2, b)

            transpose(tb[b], ob[b], _TW)
            fire_out(c, b)
        return 0

    lax.fori_loop(0, _TPW // 2, outer, 0)
    if _TPW % 2:
        c_last = base + _TPW - 1
        wait_in(c_last, 0)
        wait_out(c_last - 2, 0)
        transpose(tb0, ob0, _TW)
        fire_out(c_last, 0)
        wait_out(c_last - 1, 1)
        wait_out(c_last, 0)
    else:
        wait_out(base + _TPW - 2, 0)
        wait_out(base + _TPW - 1, 1)

    for w in range(_TEXTRA):
        @pl.when(wid == w)
        def _():
            c = _TPW * _NW + w
            pltpu.sync_copy(tabt_hbm.at[:, pl.ds(c * _TW, _TW)], tb0)
            transpose(tb0, ob0, _TW)
            pltpu.sync_copy(
                ob0, out_hbm.at[pl.ds(c * _TW * _D, _TW * _D)])

    @pl.when(wid == _TEXTRA)
    def _():
        pltpu.sync_copy(tabt_hbm.at[:, pl.ds(_TCOLS * _TW, _TAIL)], tbt)
        transpose(tbt, obt, _TAIL)
        pltpu.sync_copy(
            obt, out_hbm.at[pl.ds(_TCOLS * _TW * _D, _TAIL * _D)])


_transp = pl.kernel(
    _transp_body,
    out_type=jax.ShapeDtypeStruct((_VOCAB * _D,), jnp.float32),
    mesh=plsc.VectorSubcoreMesh(
        core_axis_name="c", subcore_axis_name="s", num_cores=_NC,
        num_subcores=_NS),
    scratch_types=[
        pltpu.VMEM((_D, _TW), jnp.float32),
        pltpu.VMEM((_D, _TW), jnp.float32),
        pltpu.VMEM((_TW * _D,), jnp.float32),
        pltpu.VMEM((_TW * _D,), jnp.float32),
        pltpu.VMEM((_D, _TAIL), jnp.float32),
        pltpu.VMEM((_TAIL * _D,), jnp.float32),
        pltpu.SemaphoreType.DMA((2,)),
        pltpu.SemaphoreType.DMA((2,)),
    ],
    compiler_params=pltpu.CompilerParams(
        use_tc_tiling_on_sc=True, needs_layout_passes=False),
)


def _gather_body(fidx_hbm, table_hbm, out_hbm, idx0, idx1, rows0, rows1,
                 t0, t1, g_sem, o_sem):
    wid = lax.axis_index("s") * _NC + lax.axis_index("c")
    idx = (idx0, idx1)
    rows = (rows0, rows1)
    tbuf = (t0, t1)
    lanes = jax.lax.broadcasted_iota(jnp.int32, (16,), 0)

    def unit_off(i):
        u = wid * _PER_W + i
        f = u // _NBLK
        b0 = (u % _NBLK) * _BLK
        return f, b0

    def load_and_fire(i, b):
        f, b0 = unit_off(i)
        pltpu.sync_copy(fidx_hbm.at[pl.ds(f * _BATCH + b0, _BLK)], idx[b])
        pltpu.make_async_copy(table_hbm.at[idx[b]], rows[b],
                              g_sem.at[b]).start()

    def wait_gather(b):
        pltpu.make_async_copy(table_hbm.at[idx[b]], rows[b],
                              g_sem.at[b]).wait()

    rot = [(lanes + k) & 15 for k in range(16)]

    def transpose(b):
        src = rows[b]
        dst = tbuf[b]

        def tile_step(t, _):
            c0 = (t % (_BLK // 16)) * 16
            d0 = (t // (_BLK // 16)) * 16
            colv = lanes + c0
            for k in range(16):
                rv = rot[k] + d0
                vals = plsc.load_gather(src, [colv, rv])
                plsc.store_scatter(dst, [rv, colv], vals)
            return 0

        lax.fori_loop(0, (_BLK // 16) * (_D // 16), tile_step, 0)

    def writeback(i, b):
        f, b0 = unit_off(i)
        pltpu.make_async_copy(
            tbuf[b], out_hbm.at[pl.ds(f * _D, _D), pl.ds(b0, _BLK)],
            o_sem.at[b]).start()

    def wait_writeback(i, b):
        f, b0 = unit_off(i)
        pltpu.make_async_copy(
            tbuf[b], out_hbm.at[pl.ds(f * _D, _D), pl.ds(b0, _BLK)],
            o_sem.at[b]).wait()

    load_and_fire(0, 0)
    for i in range(_PER_W):
        b = i % 2
        if i >= 2:
            wait_writeback(i - 2, b)
        wait_gather(b)
        if i + 1 < _PER_W:
            load_and_fire(i + 1, 1 - b)
        transpose(b)
        writeback(i, b)
    wait_writeback(_PER_W - 2, _PER_W % 2)
    wait_writeback(_PER_W - 1, (_PER_W - 1) % 2)


_gather = pl.kernel(
    _gather_body,
    out_type=jax.ShapeDtypeStruct((_FIELDS * _D, _BATCH), jnp.float32),
    mesh=plsc.VectorSubcoreMesh(
        core_axis_name="c", subcore_axis_name="s", num_cores=_NC,
        num_subcores=_NS),
    scratch_types=[
        pltpu.VMEM((_BLK,), jnp.int32),
        pltpu.VMEM((_BLK,), jnp.int32),
        pltpu.VMEM((_BLK, _D), jnp.float32),
        pltpu.VMEM((_BLK, _D), jnp.float32),
        pltpu.VMEM((_D, _BLK), jnp.float32),
        pltpu.VMEM((_D, _BLK), jnp.float32),
        pltpu.SemaphoreType.DMA((2,)),
        pltpu.SemaphoreType.DMA((2,)),
    ],
    compiler_params=pltpu.CompilerParams(
        use_tc_tiling_on_sc=False, needs_layout_passes=False),
)


@jax.jit
def kernel(inputs, table):
    fidx = inputs.T.reshape(_N).astype(jnp.int32)
    table_lin = _transp(table.T).reshape(_VOCAB, _D)
    out2 = _gather(fidx, table_lin)
    return out2.reshape(_FIELDS, _D, _BATCH).transpose(2, 0, 1)


# speedup vs baseline: 1.7390x; 1.7390x over previous
import jax
import jax.numpy as jnp
from jax import lax
from jax.experimental import pallas as pl
from jax.experimental.pallas import tpu as pltpu
from jax.experimental.pallas import tpu_sc as plsc

_VOCAB = 1000000
_D = 32
_BATCH = 16384
_FIELDS = 26
_N = _BATCH * _FIELDS
_NC = 2
_NS = 16
_NW = _NC * _NS
_BLK = 512
_NBLK = _BATCH // _BLK
_UNITS = _FIELDS * _NBLK
_PER_W = _UNITS // _NW

_TW = 512
_TCOLS = _VOCAB // _TW
_TPW = _TCOLS // _NW
_TEXTRA = _TCOLS - _TPW * _NW
_TAIL = _VOCAB - _TCOLS * _TW


def _transp_body(tabt_hbm, out_hbm, tb0, tb1, ob0, ob1, tbt, obt, i_sem,
                 o_sem):
    wid = lax.axis_index("s") * _NC + lax.axis_index("c")
    base = wid * _TPW
    lanes = lax.broadcasted_iota(jnp.int32, (16,), 0)
    lanes32 = lanes * 32
    rot = [(lanes + k) & 15 for k in range(16)]
    tb = (tb0, tb1)
    ob = (ob0, ob1)

    def transpose(src, dst, ncols):
        @plsc.parallel_loop(0, (ncols // 16) * (_D // 16), unroll=2)
        def tile_step(t):
            c0 = (t % (ncols // 16)) * 16
            d0 = (t // (ncols // 16)) * 16
            colv = lanes + c0
            dv0 = lanes32 + (c0 * 32 + d0)
            for k in range(16):
                rv = rot[k] + d0
                vals = plsc.load_gather(src, [rv, colv])
                plsc.store_scatter(dst, [dv0 + rot[k]], vals)

    def fire_in(c, b):
        pltpu.make_async_copy(
            tabt_hbm.at[:, pl.ds(c * _TW, _TW)], tb[b], i_sem.at[b]).start()

    def wait_in(c, b):
        pltpu.make_async_copy(
            tabt_hbm.at[:, pl.ds(c * _TW, _TW)], tb[b], i_sem.at[b]).wait()

    def fire_out(c, b):
        pltpu.make_async_copy(
            ob[b], out_hbm.at[pl.ds(c * _TW * _D, _TW * _D)], o_sem.at[b]).start()

    def wait_out(c, b):
        pltpu.make_async_copy(
            ob[b], out_hbm.at[pl.ds(c * _TW * _D, _TW * _D)], o_sem.at[b]).wait()

    fire_in(base, 0)

    def outer(i2, _):
        for b in range(2):
            j = i2 * 2 + b
            c = base + j
            wait_in(c, b)

            @pl.when(j + 1 < _TPW)
            def _():
                fire_in(c + 1, 1 - b)

            @pl.when(j >= 2)
            def _():
                wait_out(c - 2, b)

            transpose(tb[b], ob[b], _TW)
            fire_out(c, b)
        return 0

    lax.fori_loop(0, _TPW // 2, outer, 0)
    if _TPW % 2:
        c_last = base + _TPW - 1
        wait_in(c_last, 0)
        wait_out(c_last - 2, 0)
        transpose(tb0, ob0, _TW)
        fire_out(c_last, 0)
        wait_out(c_last - 1, 1)
        wait_out(c_last, 0)
    else:
        wait_out(base + _TPW - 2, 0)
        wait_out(base + _TPW - 1, 1)

    for w in range(_TEXTRA):
        @pl.when(wid == w)
        def _():
            c = _TPW * _NW + w
            pltpu.sync_copy(tabt_hbm.at[:, pl.ds(c * _TW, _TW)], tb0)
            transpose(tb0, ob0, _TW)
            pltpu.sync_copy(
                ob0, out_hbm.at[pl.ds(c * _TW * _D, _TW * _D)])

    @pl.when(wid == _TEXTRA)
    def _():
        pltpu.sync_copy(tabt_hbm.at[:, pl.ds(_TCOLS * _TW, _TAIL)], tbt)
        transpose(tbt, obt, _TAIL)
        pltpu.sync_copy(
            obt, out_hbm.at[pl.ds(_TCOLS * _TW * _D, _TAIL * _D)])


_transp = pl.kernel(
    _transp_body,
    out_type=jax.ShapeDtypeStruct((_VOCAB * _D,), jnp.float32),
    mesh=plsc.VectorSubcoreMesh(
        core_axis_name="c", subcore_axis_name="s", num_cores=_NC,
        num_subcores=_NS),
    scratch_types=[
        pltpu.VMEM((_D, _TW), jnp.float32),
        pltpu.VMEM((_D, _TW), jnp.float32),
        pltpu.VMEM((_TW * _D,), jnp.float32),
        pltpu.VMEM((_TW * _D,), jnp.float32),
        pltpu.VMEM((_D, _TAIL), jnp.float32),
        pltpu.VMEM((_TAIL * _D,), jnp.float32),
        pltpu.SemaphoreType.DMA((2,)),
        pltpu.SemaphoreType.DMA((2,)),
    ],
    compiler_params=pltpu.CompilerParams(
        use_tc_tiling_on_sc=True, needs_layout_passes=False),
)


def _gather_body(fidx_hbm, table_hbm, out_hbm, idx0, idx1, rows0, rows1,
                 t0, t1, g_sem, o_sem):
    wid = lax.axis_index("s") * _NC + lax.axis_index("c")
    idx = (idx0, idx1)
    rows = (rows0, rows1)
    tbuf = (t0, t1)
    lanes = jax.lax.broadcasted_iota(jnp.int32, (16,), 0)

    def unit_off(i):
        u = wid * _PER_W + i
        f = u // _NBLK
        b0 = (u % _NBLK) * _BLK
        return f, b0

    def load_and_fire(i, b):
        f, b0 = unit_off(i)
        pltpu.sync_copy(fidx_hbm.at[pl.ds(f * _BATCH + b0, _BLK)], idx[b])
        pltpu.make_async_copy(table_hbm.at[idx[b]], rows[b],
                              g_sem.at[b]).start()

    def wait_gather(b):
        pltpu.make_async_copy(table_hbm.at[idx[b]], rows[b],
                              g_sem.at[b]).wait()

    rot = [(lanes + k) & 15 for k in range(16)]

    def transpose(b):
        src = rows[b]
        dst = tbuf[b]

        @plsc.parallel_loop(0, (_BLK // 16) * (_D // 16), unroll=2)
        def tile_step(t):
            c0 = (t % (_BLK // 16)) * 16
            d0 = (t // (_BLK // 16)) * 16
            colv = lanes + c0
            for k in range(16):
                rv = rot[k] + d0
                vals = plsc.load_gather(src, [colv, rv])
                plsc.store_scatter(dst, [rv, colv], vals)

    def writeback(i, b):
        f, b0 = unit_off(i)
        pltpu.make_async_copy(
            tbuf[b], out_hbm.at[pl.ds(f * _D, _D), pl.ds(b0, _BLK)],
            o_sem.at[b]).start()

    def wait_writeback(i, b):
        f, b0 = unit_off(i)
        pltpu.make_async_copy(
            tbuf[b], out_hbm.at[pl.ds(f * _D, _D), pl.ds(b0, _BLK)],
            o_sem.at[b]).wait()

    load_and_fire(0, 0)
    for i in range(_PER_W):
        b = i % 2
        if i >= 2:
            wait_writeback(i - 2, b)
        wait_gather(b)
        if i + 1 < _PER_W:
            load_and_fire(i + 1, 1 - b)
        transpose(b)
        writeback(i, b)
    wait_writeback(_PER_W - 2, _PER_W % 2)
    wait_writeback(_PER_W - 1, (_PER_W - 1) % 2)


_gather = pl.kernel(
    _gather_body,
    out_type=jax.ShapeDtypeStruct((_FIELDS * _D, _BATCH), jnp.float32),
    mesh=plsc.VectorSubcoreMesh(
        core_axis_name="c", subcore_axis_name="s", num_cores=_NC,
        num_subcores=_NS),
    scratch_types=[
        pltpu.VMEM((_BLK,), jnp.int32),
        pltpu.VMEM((_BLK,), jnp.int32),
        pltpu.VMEM((_BLK, _D), jnp.float32),
        pltpu.VMEM((_BLK, _D), jnp.float32),
        pltpu.VMEM((_D, _BLK), jnp.float32),
        pltpu.VMEM((_D, _BLK), jnp.float32),
        pltpu.SemaphoreType.DMA((2,)),
        pltpu.SemaphoreType.DMA((2,)),
    ],
    compiler_params=pltpu.CompilerParams(
        use_tc_tiling_on_sc=False, needs_layout_passes=False),
)


@jax.jit
def kernel(inputs, table):
    fidx = inputs.T.reshape(_N).astype(jnp.int32)
    table_lin = _transp(table.T).reshape(_VOCAB, _D)
    out2 = _gather(fidx, table_lin)
    return out2.reshape(_FIELDS, _D, _BATCH).transpose(2, 0, 1)


# speedup vs baseline: 1.8754x; 1.0785x over previous
import jax
import jax.numpy as jnp
from jax import lax
from jax.experimental import pallas as pl
from jax.experimental.pallas import tpu as pltpu
from jax.experimental.pallas import tpu_sc as plsc

_VOCAB = 1000000
_D = 32
_BATCH = 16384
_FIELDS = 26
_N = _BATCH * _FIELDS
_NC = 2
_NS = 16
_NW = _NC * _NS
_BLK = 512
_NBLK = _BATCH // _BLK
_UNITS = _FIELDS * _NBLK
_PER_W = _UNITS // _NW

_TW = 512
_TCOLS = _VOCAB // _TW
_TPW = _TCOLS // _NW
_TEXTRA = _TCOLS - _TPW * _NW
_TAIL = _VOCAB - _TCOLS * _TW


def _transp_body(tabt_hbm, out_hbm, tb0, tb1, ob0, ob1, tbt, obt, i_sem,
                 o_sem):
    wid = lax.axis_index("s") * _NC + lax.axis_index("c")
    base = wid * _TPW
    lanes = lax.broadcasted_iota(jnp.int32, (16,), 0)
    lanes32 = lanes * 32
    rot = [(lanes + k) & 15 for k in range(16)]
    tb = (tb0, tb1)
    ob = (ob0, ob1)

    def transpose(src, dst, ncols):
        @plsc.parallel_loop(0, (ncols // 16) * (_D // 16), unroll=2)
        def tile_step(t):
            c0 = (t % (ncols // 16)) * 16
            d0 = (t // (ncols // 16)) * 16
            colv = lanes + c0
            dv0 = lanes32 + (c0 * 32 + d0)
            for k in range(16):
                rv = rot[k] + d0
                vals = plsc.load_gather(src, [rv, colv])
                plsc.store_scatter(dst, [dv0 + rot[k]], vals)

    def fire_in(c, b):
        pltpu.make_async_copy(
            tabt_hbm.at[:, pl.ds(c * _TW, _TW)], tb[b], i_sem.at[b]).start()

    def wait_in(c, b):
        pltpu.make_async_copy(
            tabt_hbm.at[:, pl.ds(c * _TW, _TW)], tb[b], i_sem.at[b]).wait()

    def fire_out(c, b):
        pltpu.make_async_copy(
            ob[b], out_hbm.at[pl.ds(c * _TW * _D, _TW * _D)], o_sem.at[b]).start()

    def wait_out(c, b):
        pltpu.make_async_copy(
            ob[b], out_hbm.at[pl.ds(c * _TW * _D, _TW * _D)], o_sem.at[b]).wait()

    fire_in(base, 0)

    def outer(i2, _):
        for b in range(2):
            j = i2 * 2 + b
            c = base + j
            wait_in(c, b)

            @pl.when(j + 1 < _TPW)
            def _():
                fire_in(c + 1, 1 - b)

            @pl.when(j >= 2)
            def _():
                wait_out(c - 2, b)

            transpose(tb[b], ob[b], _TW)
            fire_out(c, b)
        return 0

    lax.fori_loop(0, _TPW // 2, outer, 0)
    if _TPW % 2:
        c_last = base + _TPW - 1
        wait_in(c_last, 0)
        wait_out(c_last - 2, 0)
        transpose(tb0, ob0, _TW)
        fire_out(c_last, 0)
        wait_out(c_last - 1, 1)
        wait_out(c_last, 0)
    else:
        wait_out(base + _TPW - 2, 0)
        wait_out(base + _TPW - 1, 1)

    for w in range(_TEXTRA):
        @pl.when(wid == w)
        def _():
            c = _TPW * _NW + w
            pltpu.sync_copy(tabt_hbm.at[:, pl.ds(c * _TW, _TW)], tb0)
            transpose(tb0, ob0, _TW)
            pltpu.sync_copy(
                ob0, out_hbm.at[pl.ds(c * _TW * _D, _TW * _D)])

    @pl.when(wid == _TEXTRA)
    def _():
        pltpu.sync_copy(tabt_hbm.at[:, pl.ds(_TCOLS * _TW, _TAIL)], tbt)
        transpose(tbt, obt, _TAIL)
        pltpu.sync_copy(
            obt, out_hbm.at[pl.ds(_TCOLS * _TW * _D, _TAIL * _D)])


_transp = pl.kernel(
    _transp_body,
    out_type=jax.ShapeDtypeStruct((_VOCAB * _D,), jnp.float32),
    mesh=plsc.VectorSubcoreMesh(
        core_axis_name="c", subcore_axis_name="s", num_cores=_NC,
        num_subcores=_NS),
    scratch_types=[
        pltpu.VMEM((_D, _TW), jnp.float32),
        pltpu.VMEM((_D, _TW), jnp.float32),
        pltpu.VMEM((_TW * _D,), jnp.float32),
        pltpu.VMEM((_TW * _D,), jnp.float32),
        pltpu.VMEM((_D, _TAIL), jnp.float32),
        pltpu.VMEM((_TAIL * _D,), jnp.float32),
        pltpu.SemaphoreType.DMA((2,)),
        pltpu.SemaphoreType.DMA((2,)),
    ],
    compiler_params=pltpu.CompilerParams(
        use_tc_tiling_on_sc=True, needs_layout_passes=False),
)


def _gather_body(fidx_hbm, table_hbm, out_hbm, idxall, rows0, rows1, rows2,
                 t0, t1, g_sem, o_sem):
    wid = lax.axis_index("s") * _NC + lax.axis_index("c")
    b0 = wid * _BLK
    rows = (rows0, rows1, rows2)
    tbuf = (t0, t1)
    lanes = jax.lax.broadcasted_iota(jnp.int32, (16,), 0)

    pltpu.sync_copy(fidx_hbm.at[:, pl.ds(b0, _BLK)], idxall)

    def fire_gather(i):
        pltpu.make_async_copy(table_hbm.at[idxall.at[i]], rows[i % 3],
                              g_sem.at[i % 3]).start()

    def wait_gather(i):
        pltpu.make_async_copy(table_hbm.at[idxall.at[i]], rows[i % 3],
                              g_sem.at[i % 3]).wait()

    rot = [(lanes + k) & 15 for k in range(16)]

    def transpose(i, b2):
        src = rows[i % 3]
        dst = tbuf[b2]

        @plsc.parallel_loop(0, (_BLK // 16) * (_D // 16), unroll=2)
        def tile_step(t):
            c0 = (t % (_BLK // 16)) * 16
            d0 = (t // (_BLK // 16)) * 16
            colv = lanes + c0
            for k in range(16):
                rv = rot[k] + d0
                vals = plsc.load_gather(src, [colv, rv])
                plsc.store_scatter(dst, [rv, colv], vals)

    def writeback(i, b2):
        pltpu.make_async_copy(
            tbuf[b2], out_hbm.at[pl.ds(i * _D, _D), pl.ds(b0, _BLK)],
            o_sem.at[b2]).start()

    def wait_writeback(i, b2):
        pltpu.make_async_copy(
            tbuf[b2], out_hbm.at[pl.ds(i * _D, _D), pl.ds(b0, _BLK)],
            o_sem.at[b2]).wait()

    fire_gather(0)
    fire_gather(1)
    for i in range(_FIELDS):
        b2 = i % 2
        wait_gather(i)
        if i + 2 < _FIELDS:
            fire_gather(i + 2)
        if i >= 2:
            wait_writeback(i - 2, b2)
        transpose(i, b2)
        writeback(i, b2)
    wait_writeback(_FIELDS - 2, _FIELDS % 2)
    wait_writeback(_FIELDS - 1, (_FIELDS - 1) % 2)


_gather = pl.kernel(
    _gather_body,
    out_type=jax.ShapeDtypeStruct((_FIELDS * _D, _BATCH), jnp.float32),
    mesh=plsc.VectorSubcoreMesh(
        core_axis_name="c", subcore_axis_name="s", num_cores=_NC,
        num_subcores=_NS),
    scratch_types=[
        pltpu.VMEM((_FIELDS, _BLK), jnp.int32),
        pltpu.VMEM((_BLK, _D), jnp.float32),
        pltpu.VMEM((_BLK, _D), jnp.float32),
        pltpu.VMEM((_BLK, _D), jnp.float32),
        pltpu.VMEM((_D, _BLK), jnp.float32),
        pltpu.VMEM((_D, _BLK), jnp.float32),
        pltpu.SemaphoreType.DMA((3,)),
        pltpu.SemaphoreType.DMA((2,)),
    ],
    compiler_params=pltpu.CompilerParams(
        use_tc_tiling_on_sc=False, needs_layout_passes=False),
)


@jax.jit
def kernel(inputs, table):
    fidx2d = inputs.T.astype(jnp.int32)
    table_lin = _transp(table.T).reshape(_VOCAB, _D)
    out2 = _gather(fidx2d, table_lin)
    return out2.reshape(_FIELDS, _D, _BATCH).transpose(2, 0, 1)
